# baseline (speedup 1.0000x reference)
import jax
import jax.numpy as jnp
from jax import lax
from jax.experimental import pallas as pl
from jax.experimental.pallas import tpu as pltpu

BSIZE, NQUERY, SEQL, ISIZE = 32, 16, 8192, 128
NROWS = BSIZE * NQUERY
CHUNKS = (160, 160, 160, 32)
STARTS = tuple(sum(CHUNKS[:i]) for i in range(len(CHUNKS)))
NCHUNK = len(CHUNKS)


def _fused_body(x_ref, vmat_ref, out_ref, idx_vmem, idx_smem,
                rows_vmem, ssem, isem, gsem, osem, *bufs):

    def _stream_copy(c):
        return pltpu.make_async_copy(
            x_ref.at[pl.ds(STARTS[c], CHUNKS[c]), :],
            bufs[c],
            ssem.at[c],
        )

    def _issue_gathers(c):
        pltpu.make_async_copy(
            idx_vmem.at[pl.ds(STARTS[c], CHUNKS[c]), :],
            idx_smem.at[pl.ds(STARTS[c], CHUNKS[c]), :],
            isem,
        ).wait()
        for r in range(CHUNKS[c]):
            row = STARTS[c] + r
            s = idx_smem[row, 0]
            pltpu.make_async_copy(
                vmat_ref.at[pl.ds(s, 1), :],
                rows_vmem.at[pl.ds(row, 1), :],
                gsem,
            ).start()

    for c in range(NCHUNK):
        _stream_copy(c).start()

    for c in range(NCHUNK):
        _stream_copy(c).wait()
        x = bufs[c][...]
        m = jnp.max(x, axis=1, keepdims=True)
        col = lax.broadcasted_iota(jnp.int32, x.shape, 1)
        idx = jnp.min(jnp.where(x == m, col, jnp.int32(SEQL)), axis=1,
                      keepdims=True)
        rows = STARTS[c] + lax.broadcasted_iota(jnp.int32, (CHUNKS[c], 1), 0)
        flat = idx + (rows // NQUERY) * SEQL
        idx_vmem[pl.ds(STARTS[c], CHUNKS[c]), :] = flat
        pltpu.make_async_copy(
            idx_vmem.at[pl.ds(STARTS[c], CHUNKS[c]), :],
            idx_smem.at[pl.ds(STARTS[c], CHUNKS[c]), :],
            isem,
        ).start()
        if c > 0:
            _issue_gathers(c - 1)

    _issue_gathers(NCHUNK - 1)
    pltpu.make_async_copy(
        vmat_ref.at[pl.ds(0, NROWS), :], rows_vmem, gsem,
    ).wait()
    pltpu.make_async_copy(rows_vmem, out_ref, osem).start()
    pltpu.make_async_copy(rows_vmem, out_ref, osem).wait()


_fused_call = pl.pallas_call(
    _fused_body,
    in_specs=[
        pl.BlockSpec(memory_space=pltpu.MemorySpace.HBM),
        pl.BlockSpec(memory_space=pltpu.MemorySpace.HBM),
    ],
    out_specs=pl.BlockSpec(memory_space=pltpu.MemorySpace.HBM),
    out_shape=jax.ShapeDtypeStruct((NROWS, ISIZE), jnp.float32),
    scratch_shapes=[
        pltpu.VMEM((NROWS, 1), jnp.int32),
        pltpu.SMEM((NROWS, 1), jnp.int32),
        pltpu.VMEM((NROWS, ISIZE), jnp.float32),
        pltpu.SemaphoreType.DMA((NCHUNK,)),
        pltpu.SemaphoreType.DMA,
        pltpu.SemaphoreType.DMA,
        pltpu.SemaphoreType.DMA,
    ] + [pltpu.VMEM((sz, SEQL), jnp.float32) for sz in CHUNKS],
)


def kernel(attnmat, vmat):
    bsize, nquery, seql = attnmat.shape
    isize = vmat.shape[-1]
    attn2d = attnmat.reshape(bsize * nquery, seql)
    flat_v = vmat.reshape(bsize * seql, isize)
    out = _fused_call(attn2d, flat_v)
    return out.reshape(bsize, nquery, isize)


# speedup vs baseline: 1.0170x; 1.0170x over previous
import jax
import jax.numpy as jnp
from jax import lax
from jax.experimental import pallas as pl
from jax.experimental.pallas import tpu as pltpu

BSIZE, NQUERY, SEQL, ISIZE = 32, 16, 8192, 128
NROWS = BSIZE * NQUERY
CHUNKS = (128, 128, 128, 128)
STARTS = tuple(sum(CHUNKS[:i]) for i in range(len(CHUNKS)))
NCHUNK = len(CHUNKS)


def _fused_body(x_ref, vmat_ref, out_ref, idx_vmem, idx_smem,
                rows_vmem, ssem, isem, gsem, osem, *bufs):

    def _stream_copy(c):
        return pltpu.make_async_copy(
            x_ref.at[pl.ds(STARTS[c], CHUNKS[c]), :],
            bufs[c],
            ssem.at[c],
        )

    def _issue_gathers(c):
        pltpu.make_async_copy(
            idx_vmem.at[pl.ds(STARTS[c], CHUNKS[c]), :],
            idx_smem.at[pl.ds(STARTS[c], CHUNKS[c]), :],
            isem,
        ).wait()
        for r in range(CHUNKS[c]):
            row = STARTS[c] + r
            s = idx_smem[row, 0]
            pltpu.make_async_copy(
                vmat_ref.at[pl.ds(s, 1), :],
                rows_vmem.at[pl.ds(row, 1), :],
                gsem,
            ).start()

    for c in range(NCHUNK):
        _stream_copy(c).start()

    for c in range(NCHUNK):
        _stream_copy(c).wait()
        x = bufs[c][...]
        m = jnp.max(x, axis=1, keepdims=True)
        col = lax.broadcasted_iota(jnp.int32, x.shape, 1)
        idx = jnp.min(jnp.where(x == m, col, jnp.int32(SEQL)), axis=1,
                      keepdims=True)
        rows = STARTS[c] + lax.broadcasted_iota(jnp.int32, (CHUNKS[c], 1), 0)
        flat = idx + (rows // NQUERY) * SEQL
        idx_vmem[pl.ds(STARTS[c], CHUNKS[c]), :] = flat
        pltpu.make_async_copy(
            idx_vmem.at[pl.ds(STARTS[c], CHUNKS[c]), :],
            idx_smem.at[pl.ds(STARTS[c], CHUNKS[c]), :],
            isem,
        ).start()
        if c > 0:
            _issue_gathers(c - 1)

    _issue_gathers(NCHUNK - 1)
    pltpu.make_async_copy(
        vmat_ref.at[pl.ds(0, NROWS), :], rows_vmem, gsem,
    ).wait()
    pltpu.make_async_copy(rows_vmem, out_ref, osem).start()
    pltpu.make_async_copy(rows_vmem, out_ref, osem).wait()


_fused_call = pl.pallas_call(
    _fused_body,
    in_specs=[
        pl.BlockSpec(memory_space=pltpu.MemorySpace.HBM),
        pl.BlockSpec(memory_space=pltpu.MemorySpace.HBM),
    ],
    out_specs=pl.BlockSpec(memory_space=pltpu.MemorySpace.HBM),
    out_shape=jax.ShapeDtypeStruct((NROWS, ISIZE), jnp.float32),
    scratch_shapes=[
        pltpu.VMEM((NROWS, 1), jnp.int32),
        pltpu.SMEM((NROWS, 1), jnp.int32),
        pltpu.VMEM((NROWS, ISIZE), jnp.float32),
        pltpu.SemaphoreType.DMA((NCHUNK,)),
        pltpu.SemaphoreType.DMA,
        pltpu.SemaphoreType.DMA,
        pltpu.SemaphoreType.DMA,
    ] + [pltpu.VMEM((sz, SEQL), jnp.float32) for sz in CHUNKS],
)


def kernel(attnmat, vmat):
    bsize, nquery, seql = attnmat.shape
    isize = vmat.shape[-1]
    attn2d = attnmat.reshape(bsize * nquery, seql)
    flat_v = vmat.reshape(bsize * seql, isize)
    out = _fused_call(attn2d, flat_v)
    return out.reshape(bsize, nquery, isize)
